# Initial kernel scaffold; baseline (speedup 1.0000x reference)
import functools

import jax
import jax.numpy as jnp
from jax import lax
from jax.experimental import pallas as pl
from jax.experimental.pallas import tpu as pltpu
from jax.experimental.pallas import tpu_sc as plsc

N = 10000
E = 160000
D = 256
H = 64
C = 64

NC = 2
NS = 16
CHUNK = 128
E_PAD = 163840
N_PAD = 10240
CHUNKS_PER_TILE = (E_PAD // CHUNK) // (NC * NS)
ROWS_PER_TILE = N_PAD // NS

_MESH = dict(core_axis_name="c", subcore_axis_name="s")


def _sc_degree(dst2d, zeros_n):
    mesh = plsc.VectorSubcoreMesh(**_MESH)

    @functools.partial(
        pl.kernel,
        out_type=jax.ShapeDtypeStruct((NC, N_PAD), jnp.float32),
        mesh=mesh,
        scratch_types=[
            pltpu.VMEM((CHUNKS_PER_TILE, CHUNK), jnp.int32),
            pltpu.VMEM((CHUNK,), jnp.float32),
            pltpu.VMEM_SHARED((N_PAD,), jnp.float32),
        ],
    )
    def deg_kernel(dst_hbm, zeros_hbm, out_hbm, idx_v, ones_v, acc_sh):
        cid = lax.axis_index("c")
        sid = lax.axis_index("s")
        wid = cid * NS + sid
        base = wid * CHUNKS_PER_TILE
        pltpu.sync_copy(dst_hbm.at[pl.ds(base, CHUNKS_PER_TILE)], idx_v)

        @pl.loop(0, CHUNK, step=16)
        def _(i):
            ones_v[pl.ds(i, 16)] = jnp.ones((16,), jnp.float32)

        r0 = sid * ROWS_PER_TILE
        pltpu.sync_copy(zeros_hbm.at[pl.ds(r0, ROWS_PER_TILE)],
                        acc_sh.at[pl.ds(r0, ROWS_PER_TILE)])
        plsc.subcore_barrier()

        @pl.loop(0, CHUNKS_PER_TILE)
        def _(j):
            pltpu.sync_copy(ones_v, acc_sh.at[idx_v.at[j]], add=True)

        plsc.subcore_barrier()
        pltpu.sync_copy(acc_sh.at[pl.ds(r0, ROWS_PER_TILE)],
                        out_hbm.at[cid, pl.ds(r0, ROWS_PER_TILE)])

    return deg_kernel(dst2d, zeros_n)


def _sc_aggregate(g, src2d, dst2d, zeros_rows):
    mesh = plsc.VectorSubcoreMesh(**_MESH)

    @functools.partial(
        pl.kernel,
        out_type=jax.ShapeDtypeStruct((NC, N_PAD, H), jnp.float32),
        mesh=mesh,
        scratch_types=[
            pltpu.VMEM((CHUNKS_PER_TILE, CHUNK), jnp.int32),
            pltpu.VMEM((CHUNKS_PER_TILE, CHUNK), jnp.int32),
            pltpu.VMEM((CHUNK, H), jnp.float32),
            pltpu.VMEM_SHARED((N_PAD, H), jnp.float32),
            pltpu.SemaphoreType.DMA,
        ],
    )
    def agg_kernel(g_hbm, src_hbm, dst_hbm, zeros_hbm, out_hbm,
                   src_v, dst_v, rows_v, acc_sh, sem):
        cid = lax.axis_index("c")
        sid = lax.axis_index("s")
        wid = cid * NS + sid
        base = wid * CHUNKS_PER_TILE
        pltpu.sync_copy(src_hbm.at[pl.ds(base, CHUNKS_PER_TILE)], src_v)
        pltpu.sync_copy(dst_hbm.at[pl.ds(base, CHUNKS_PER_TILE)], dst_v)

        r0 = sid * ROWS_PER_TILE
        pltpu.sync_copy(zeros_hbm.at[pl.ds(r0, ROWS_PER_TILE)],
                        acc_sh.at[pl.ds(r0, ROWS_PER_TILE)])
        plsc.subcore_barrier()

        @pl.loop(0, CHUNKS_PER_TILE)
        def _(j):
            pltpu.async_copy(g_hbm.at[src_v.at[j]], rows_v, sem).wait()
            pltpu.sync_copy(rows_v, acc_sh.at[dst_v.at[j]], add=True)

        plsc.subcore_barrier()
        pltpu.sync_copy(acc_sh.at[pl.ds(r0, ROWS_PER_TILE)],
                        out_hbm.at[cid, pl.ds(r0, ROWS_PER_TILE)])

    return agg_kernel(g, src2d, dst2d, zeros_rows)


_BM = 1000


def _dot(a, b):
    return lax.dot_general(a, b, (((1,), (0,)), ((), ())),
                           precision=lax.Precision.HIGHEST,
                           preferred_element_type=jnp.float32)


def _tc_matmul(x, w):
    m, k = x.shape
    _, h = w.shape

    def body(x_ref, w_ref, o_ref):
        o_ref[...] = _dot(x_ref[...], w_ref[...])

    return pl.pallas_call(
        body,
        grid=(m // _BM,),
        in_specs=[pl.BlockSpec((_BM, k), lambda i: (i, 0)),
                  pl.BlockSpec((k, h), lambda i: (0, 0))],
        out_specs=pl.BlockSpec((_BM, h), lambda i: (i, 0)),
        out_shape=jax.ShapeDtypeStruct((m, h), jnp.float32),
    )(x, w)


def _dinv_of(d_ref):
    return lax.rsqrt(d_ref[0, :] + d_ref[1, :] + 1.0)


def _tc_scale(h, degp):
    def body(h_ref, d_ref, o_ref):
        o_ref[...] = h_ref[...] * _dinv_of(d_ref)[:, None]

    return pl.pallas_call(
        body,
        grid=(N // _BM,),
        in_specs=[pl.BlockSpec((_BM, H), lambda i: (i, 0)),
                  pl.BlockSpec((2, _BM), lambda i: (0, i))],
        out_specs=pl.BlockSpec((_BM, H), lambda i: (i, 0)),
        out_shape=jax.ShapeDtypeStruct((N, H), jnp.float32),
    )(h, degp)


def _tc_mid(aggp, g0, degp, b0, w1):
    def body(a_ref, g_ref, d_ref, b_ref, w_ref, o_ref):
        dinv = _dinv_of(d_ref)[:, None]
        s = (a_ref[0] + a_ref[1] + g_ref[...]) * dinv + b_ref[...]
        o_ref[...] = _dot(jnp.maximum(s, 0.0), w_ref[...]) * dinv

    return pl.pallas_call(
        body,
        grid=(N // _BM,),
        in_specs=[pl.BlockSpec((2, _BM, H), lambda i: (0, i, 0)),
                  pl.BlockSpec((_BM, H), lambda i: (i, 0)),
                  pl.BlockSpec((2, _BM), lambda i: (0, i)),
                  pl.BlockSpec((1, H), lambda i: (0, 0)),
                  pl.BlockSpec((H, C), lambda i: (0, 0))],
        out_specs=pl.BlockSpec((_BM, C), lambda i: (i, 0)),
        out_shape=jax.ShapeDtypeStruct((N, C), jnp.float32),
    )(aggp, g0, degp, b0, w1)


def _tc_final(aggp, g1, degp, b1):
    def body(a_ref, g_ref, d_ref, b_ref, o_ref):
        dinv = _dinv_of(d_ref)[:, None]
        o_ref[...] = (a_ref[0] + a_ref[1] + g_ref[...]) * dinv + b_ref[...]

    return pl.pallas_call(
        body,
        grid=(N // _BM,),
        in_specs=[pl.BlockSpec((2, _BM, C), lambda i: (0, i, 0)),
                  pl.BlockSpec((_BM, C), lambda i: (i, 0)),
                  pl.BlockSpec((2, _BM), lambda i: (0, i)),
                  pl.BlockSpec((1, C), lambda i: (0, 0))],
        out_specs=pl.BlockSpec((_BM, C), lambda i: (i, 0)),
        out_shape=jax.ShapeDtypeStruct((N, C), jnp.float32),
    )(aggp, g1, degp, b1)


def kernel(x, edge_index, W0, b0, W1, b1):
    src = edge_index[0].astype(jnp.int32)
    dst = edge_index[1].astype(jnp.int32)
    npad = E_PAD - E
    src2d = jnp.concatenate(
        [src, jnp.zeros((npad,), jnp.int32)]).reshape(E_PAD // CHUNK, CHUNK)
    dst2d = jnp.concatenate(
        [dst, jnp.full((npad,), N_PAD - 1, jnp.int32)]).reshape(
            E_PAD // CHUNK, CHUNK)
    zeros_rows = jnp.zeros((N_PAD, H), jnp.float32)
    zeros_n = jnp.zeros((N_PAD,), jnp.float32)

    degp = _sc_degree(dst2d, zeros_n)[:, :N]
    h0 = _tc_matmul(x, W0)
    g0 = _tc_scale(h0, degp)
    agg0 = _sc_aggregate(g0, src2d, dst2d, zeros_rows)[:, :N]
    g1 = _tc_mid(agg0, g0, degp, b0.reshape(1, H), W1)
    agg1 = _sc_aggregate(g1, src2d, dst2d, zeros_rows)[:, :N]
    return _tc_final(agg1, g1, degp, b1.reshape(1, C))


# speedup vs baseline: 11.2772x; 11.2772x over previous
import functools

import jax
import jax.numpy as jnp
from jax import lax
from jax.experimental import pallas as pl
from jax.experimental.pallas import tpu as pltpu
from jax.experimental.pallas import tpu_sc as plsc

N = 10000
E = 160000
D = 256
H = 64
C = 64

NC = 2
NS = 16
CHUNK = 128
E_PAD = 163840
N_PAD = 10240
CHUNKS_PER_TILE = (E_PAD // CHUNK) // (NC * NS)
ROWS_PER_TILE = N_PAD // NS

_MESH = dict(core_axis_name="c", subcore_axis_name="s")


def _sc_degree(dst2d, zeros_n):
    mesh = plsc.VectorSubcoreMesh(**_MESH)

    @functools.partial(
        pl.kernel,
        out_type=jax.ShapeDtypeStruct((NC, N_PAD), jnp.float32),
        mesh=mesh,
        compiler_params=pltpu.CompilerParams(use_tc_tiling_on_sc=False),
        scratch_types=[
            pltpu.VMEM((CHUNKS_PER_TILE, CHUNK), jnp.int32),
            pltpu.VMEM((CHUNK,), jnp.float32),
            pltpu.VMEM_SHARED((N_PAD,), jnp.float32),
        ],
    )
    def deg_kernel(dst_hbm, zeros_hbm, out_hbm, idx_v, ones_v, acc_sh):
        cid = lax.axis_index("c")
        sid = lax.axis_index("s")
        wid = cid * NS + sid
        base = wid * CHUNKS_PER_TILE
        pltpu.sync_copy(dst_hbm.at[pl.ds(base, CHUNKS_PER_TILE)], idx_v)

        @pl.loop(0, CHUNK, step=16)
        def _(i):
            ones_v[pl.ds(i, 16)] = jnp.ones((16,), jnp.float32)

        r0 = sid * ROWS_PER_TILE
        pltpu.sync_copy(zeros_hbm.at[pl.ds(r0, ROWS_PER_TILE)],
                        acc_sh.at[pl.ds(r0, ROWS_PER_TILE)])
        plsc.subcore_barrier()

        @pl.loop(0, CHUNKS_PER_TILE)
        def _(j):
            pltpu.sync_copy(ones_v, acc_sh.at[idx_v.at[j]], add=True)

        plsc.subcore_barrier()
        pltpu.sync_copy(acc_sh.at[pl.ds(r0, ROWS_PER_TILE)],
                        out_hbm.at[cid, pl.ds(r0, ROWS_PER_TILE)])

    return deg_kernel(dst2d, zeros_n)


def _sc_aggregate(g, src2d, dst2d, zeros_rows):
    mesh = plsc.VectorSubcoreMesh(**_MESH)

    @functools.partial(
        pl.kernel,
        out_type=jax.ShapeDtypeStruct((NC, N_PAD, H), jnp.float32),
        mesh=mesh,
        compiler_params=pltpu.CompilerParams(use_tc_tiling_on_sc=False),
        scratch_types=[
            pltpu.VMEM((CHUNKS_PER_TILE, CHUNK), jnp.int32),
            pltpu.VMEM((CHUNKS_PER_TILE, CHUNK), jnp.int32),
            pltpu.VMEM((CHUNK, H), jnp.float32),
            pltpu.VMEM_SHARED((N_PAD, H), jnp.float32),
            pltpu.SemaphoreType.DMA,
        ],
    )
    def agg_kernel(g_hbm, src_hbm, dst_hbm, zeros_hbm, out_hbm,
                   src_v, dst_v, rows_v, acc_sh, sem):
        cid = lax.axis_index("c")
        sid = lax.axis_index("s")
        wid = cid * NS + sid
        base = wid * CHUNKS_PER_TILE
        pltpu.sync_copy(src_hbm.at[pl.ds(base, CHUNKS_PER_TILE)], src_v)
        pltpu.sync_copy(dst_hbm.at[pl.ds(base, CHUNKS_PER_TILE)], dst_v)

        r0 = sid * ROWS_PER_TILE
        pltpu.sync_copy(zeros_hbm.at[pl.ds(r0, ROWS_PER_TILE)],
                        acc_sh.at[pl.ds(r0, ROWS_PER_TILE)])
        plsc.subcore_barrier()

        @pl.loop(0, CHUNKS_PER_TILE)
        def _(j):
            pltpu.async_copy(g_hbm.at[src_v.at[j]], rows_v, sem).wait()
            pltpu.sync_copy(rows_v, acc_sh.at[dst_v.at[j]], add=True)

        plsc.subcore_barrier()
        pltpu.sync_copy(acc_sh.at[pl.ds(r0, ROWS_PER_TILE)],
                        out_hbm.at[cid, pl.ds(r0, ROWS_PER_TILE)])

    return agg_kernel(g, src2d, dst2d, zeros_rows)


_BM = 1000


def _dot(a, b):
    return lax.dot_general(a, b, (((1,), (0,)), ((), ())),
                           precision=lax.Precision.HIGHEST,
                           preferred_element_type=jnp.float32)


def _tc_matmul(x, w):
    m, k = x.shape
    _, h = w.shape

    def body(x_ref, w_ref, o_ref):
        o_ref[...] = _dot(x_ref[...], w_ref[...])

    return pl.pallas_call(
        body,
        grid=(m // _BM,),
        in_specs=[pl.BlockSpec((_BM, k), lambda i: (i, 0)),
                  pl.BlockSpec((k, h), lambda i: (0, 0))],
        out_specs=pl.BlockSpec((_BM, h), lambda i: (i, 0)),
        out_shape=jax.ShapeDtypeStruct((m, h), jnp.float32),
    )(x, w)


def _dinv_of(d_ref):
    return lax.rsqrt(d_ref[:, 0:1] + d_ref[:, 1:2] + 1.0)


_DEG_SPEC = pl.BlockSpec((_BM, 2), lambda i: (i, 0))


def _tc_scale(h, degp):
    def body(h_ref, d_ref, o_ref):
        o_ref[...] = h_ref[...] * _dinv_of(d_ref)

    return pl.pallas_call(
        body,
        grid=(N // _BM,),
        in_specs=[pl.BlockSpec((_BM, H), lambda i: (i, 0)),
                  _DEG_SPEC],
        out_specs=pl.BlockSpec((_BM, H), lambda i: (i, 0)),
        out_shape=jax.ShapeDtypeStruct((N, H), jnp.float32),
    )(h, degp)


def _tc_mid(aggp, g0, degp, b0, w1):
    def body(a_ref, g_ref, d_ref, b_ref, w_ref, o_ref):
        dinv = _dinv_of(d_ref)
        s = (a_ref[0] + a_ref[1] + g_ref[...]) * dinv + b_ref[...]
        o_ref[...] = _dot(jnp.maximum(s, 0.0), w_ref[...]) * dinv

    return pl.pallas_call(
        body,
        grid=(N // _BM,),
        in_specs=[pl.BlockSpec((2, _BM, H), lambda i: (0, i, 0)),
                  pl.BlockSpec((_BM, H), lambda i: (i, 0)),
                  _DEG_SPEC,
                  pl.BlockSpec((1, H), lambda i: (0, 0)),
                  pl.BlockSpec((H, C), lambda i: (0, 0))],
        out_specs=pl.BlockSpec((_BM, C), lambda i: (i, 0)),
        out_shape=jax.ShapeDtypeStruct((N, C), jnp.float32),
    )(aggp, g0, degp, b0, w1)


def _tc_final(aggp, g1, degp, b1):
    def body(a_ref, g_ref, d_ref, b_ref, o_ref):
        dinv = _dinv_of(d_ref)
        o_ref[...] = (a_ref[0] + a_ref[1] + g_ref[...]) * dinv + b_ref[...]

    return pl.pallas_call(
        body,
        grid=(N // _BM,),
        in_specs=[pl.BlockSpec((2, _BM, C), lambda i: (0, i, 0)),
                  pl.BlockSpec((_BM, C), lambda i: (i, 0)),
                  _DEG_SPEC,
                  pl.BlockSpec((1, C), lambda i: (0, 0))],
        out_specs=pl.BlockSpec((_BM, C), lambda i: (i, 0)),
        out_shape=jax.ShapeDtypeStruct((N, C), jnp.float32),
    )(aggp, g1, degp, b1)


def kernel(x, edge_index, W0, b0, W1, b1):
    src = edge_index[0].astype(jnp.int32)
    dst = edge_index[1].astype(jnp.int32)
    npad = E_PAD - E
    src2d = jnp.concatenate(
        [src, jnp.zeros((npad,), jnp.int32)]).reshape(E_PAD // CHUNK, CHUNK)
    dst2d = jnp.concatenate(
        [dst, jnp.full((npad,), N_PAD - 1, jnp.int32)]).reshape(
            E_PAD // CHUNK, CHUNK)
    zeros_rows = jnp.zeros((N_PAD, H), jnp.float32)
    zeros_n = jnp.zeros((N_PAD,), jnp.float32)

    degp = _sc_degree(dst2d, zeros_n)[:, :N].T
    h0 = _tc_matmul(x, W0)
    g0 = _tc_scale(h0, degp)
    agg0 = _sc_aggregate(g0, src2d, dst2d, zeros_rows)[:, :N]
    g1 = _tc_mid(agg0, g0, degp, b0.reshape(1, H), W1)
    agg1 = _sc_aggregate(g1, src2d, dst2d, zeros_rows)[:, :N]
    return _tc_final(agg1, g1, degp, b1.reshape(1, C))


# speedup vs baseline: 13.0258x; 1.1550x over previous
import functools

import jax
import jax.numpy as jnp
from jax import lax
from jax.experimental import pallas as pl
from jax.experimental.pallas import tpu as pltpu
from jax.experimental.pallas import tpu_sc as plsc

N = 10000
E = 160000
D = 256
H = 64
C = 64

NC = 2
NS = 16
CHUNK = 128
E_PAD = 163840
N_PAD = 10240
CHUNKS_PER_TILE = (E_PAD // CHUNK) // (NC * NS)
ROWS_PER_TILE = N_PAD // NS

_MESH = dict(core_axis_name="c", subcore_axis_name="s")


def _sc_degree(dst2d, zeros_n):
    mesh = plsc.VectorSubcoreMesh(**_MESH)

    @functools.partial(
        pl.kernel,
        out_type=jax.ShapeDtypeStruct((NC, N_PAD), jnp.float32),
        mesh=mesh,
        compiler_params=pltpu.CompilerParams(use_tc_tiling_on_sc=False),
        scratch_types=[
            pltpu.VMEM((CHUNKS_PER_TILE, CHUNK), jnp.int32),
            pltpu.VMEM((CHUNK,), jnp.float32),
            pltpu.VMEM_SHARED((N_PAD,), jnp.float32),
        ],
    )
    def deg_kernel(dst_hbm, zeros_hbm, out_hbm, idx_v, ones_v, acc_sh):
        cid = lax.axis_index("c")
        sid = lax.axis_index("s")
        wid = cid * NS + sid
        base = wid * CHUNKS_PER_TILE
        pltpu.sync_copy(dst_hbm.at[pl.ds(base, CHUNKS_PER_TILE)], idx_v)

        @pl.loop(0, CHUNK, step=16)
        def _(i):
            ones_v[pl.ds(i, 16)] = jnp.ones((16,), jnp.float32)

        r0 = sid * ROWS_PER_TILE
        pltpu.sync_copy(zeros_hbm.at[pl.ds(r0, ROWS_PER_TILE)],
                        acc_sh.at[pl.ds(r0, ROWS_PER_TILE)])
        plsc.subcore_barrier()

        @pl.loop(0, CHUNKS_PER_TILE)
        def _(j):
            pltpu.sync_copy(ones_v, acc_sh.at[idx_v.at[j]], add=True)

        plsc.subcore_barrier()
        pltpu.sync_copy(acc_sh.at[pl.ds(r0, ROWS_PER_TILE)],
                        out_hbm.at[cid, pl.ds(r0, ROWS_PER_TILE)])

    return deg_kernel(dst2d, zeros_n)


def _sc_aggregate(g, src2d, dst2d, zeros_rows):
    mesh = plsc.VectorSubcoreMesh(**_MESH)

    @functools.partial(
        pl.kernel,
        out_type=jax.ShapeDtypeStruct((NC, N_PAD, H), jnp.float32),
        mesh=mesh,
        compiler_params=pltpu.CompilerParams(use_tc_tiling_on_sc=False),
        scratch_types=[
            pltpu.VMEM((CHUNKS_PER_TILE, CHUNK), jnp.int32),
            pltpu.VMEM((CHUNKS_PER_TILE, CHUNK), jnp.int32),
            pltpu.VMEM((CHUNK, H), jnp.float32),
            pltpu.VMEM((CHUNK, H), jnp.float32),
            pltpu.VMEM_SHARED((N_PAD, H), jnp.float32),
            pltpu.SemaphoreType.DMA,
            pltpu.SemaphoreType.DMA,
        ],
    )
    def agg_kernel(g_hbm, src_hbm, dst_hbm, zeros_hbm, out_hbm,
                   src_v, dst_v, rows_a, rows_b, acc_sh, sem_a, sem_b):
        cid = lax.axis_index("c")
        sid = lax.axis_index("s")
        wid = cid * NS + sid
        base = wid * CHUNKS_PER_TILE
        pltpu.sync_copy(src_hbm.at[pl.ds(base, CHUNKS_PER_TILE)], src_v)
        pltpu.sync_copy(dst_hbm.at[pl.ds(base, CHUNKS_PER_TILE)], dst_v)

        r0 = sid * ROWS_PER_TILE
        pltpu.sync_copy(zeros_hbm.at[pl.ds(r0, ROWS_PER_TILE)],
                        acc_sh.at[pl.ds(r0, ROWS_PER_TILE)])
        plsc.subcore_barrier()

        pltpu.async_copy(g_hbm.at[src_v.at[0]], rows_a, sem_a)

        @pl.loop(0, CHUNKS_PER_TILE, step=2)
        def _(j):
            pltpu.async_copy(g_hbm.at[src_v.at[j + 1]], rows_b, sem_b)
            pltpu.make_async_copy(g_hbm.at[src_v.at[j]], rows_a, sem_a).wait()
            pltpu.sync_copy(rows_a, acc_sh.at[dst_v.at[j]], add=True)

            @pl.when(j + 2 < CHUNKS_PER_TILE)
            def _():
                pltpu.async_copy(g_hbm.at[src_v.at[j + 2]], rows_a, sem_a)

            pltpu.make_async_copy(
                g_hbm.at[src_v.at[j + 1]], rows_b, sem_b).wait()
            pltpu.sync_copy(rows_b, acc_sh.at[dst_v.at[j + 1]], add=True)

        plsc.subcore_barrier()
        pltpu.sync_copy(acc_sh.at[pl.ds(r0, ROWS_PER_TILE)],
                        out_hbm.at[cid, pl.ds(r0, ROWS_PER_TILE)])

    return agg_kernel(g, src2d, dst2d, zeros_rows)


_BM = 1000


def _dot(a, b):
    return lax.dot_general(a, b, (((1,), (0,)), ((), ())),
                           precision=lax.Precision.HIGHEST,
                           preferred_element_type=jnp.float32)


def _tc_matmul(x, w):
    m, k = x.shape
    _, h = w.shape

    def body(x_ref, w_ref, o_ref):
        o_ref[...] = _dot(x_ref[...], w_ref[...])

    return pl.pallas_call(
        body,
        grid=(m // _BM,),
        in_specs=[pl.BlockSpec((_BM, k), lambda i: (i, 0)),
                  pl.BlockSpec((k, h), lambda i: (0, 0))],
        out_specs=pl.BlockSpec((_BM, h), lambda i: (i, 0)),
        out_shape=jax.ShapeDtypeStruct((m, h), jnp.float32),
    )(x, w)


def _dinv_of(d_ref):
    return lax.rsqrt(d_ref[:, 0:1] + d_ref[:, 1:2] + 1.0)


_DEG_SPEC = pl.BlockSpec((_BM, 2), lambda i: (i, 0))


def _tc_scale(h, degp):
    def body(h_ref, d_ref, o_ref):
        o_ref[...] = h_ref[...] * _dinv_of(d_ref)

    return pl.pallas_call(
        body,
        grid=(N // _BM,),
        in_specs=[pl.BlockSpec((_BM, H), lambda i: (i, 0)),
                  _DEG_SPEC],
        out_specs=pl.BlockSpec((_BM, H), lambda i: (i, 0)),
        out_shape=jax.ShapeDtypeStruct((N, H), jnp.float32),
    )(h, degp)


def _tc_mid(aggp, g0, degp, b0, w1):
    def body(a_ref, g_ref, d_ref, b_ref, w_ref, o_ref):
        dinv = _dinv_of(d_ref)
        s = (a_ref[0] + a_ref[1] + g_ref[...]) * dinv + b_ref[...]
        o_ref[...] = _dot(jnp.maximum(s, 0.0), w_ref[...]) * dinv

    return pl.pallas_call(
        body,
        grid=(N // _BM,),
        in_specs=[pl.BlockSpec((2, _BM, H), lambda i: (0, i, 0),),
                  pl.BlockSpec((_BM, H), lambda i: (i, 0)),
                  _DEG_SPEC,
                  pl.BlockSpec((1, H), lambda i: (0, 0)),
                  pl.BlockSpec((H, C), lambda i: (0, 0))],
        out_specs=pl.BlockSpec((_BM, C), lambda i: (i, 0)),
        out_shape=jax.ShapeDtypeStruct((N, C), jnp.float32),
    )(aggp, g0, degp, b0, w1)


def _tc_final(aggp, g1, degp, b1):
    def body(a_ref, g_ref, d_ref, b_ref, o_ref):
        dinv = _dinv_of(d_ref)
        o_ref[...] = (a_ref[0] + a_ref[1] + g_ref[...]) * dinv + b_ref[...]

    return pl.pallas_call(
        body,
        grid=(N // _BM,),
        in_specs=[pl.BlockSpec((2, _BM, C), lambda i: (0, i, 0)),
                  pl.BlockSpec((_BM, C), lambda i: (i, 0)),
                  _DEG_SPEC,
                  pl.BlockSpec((1, C), lambda i: (0, 0))],
        out_specs=pl.BlockSpec((_BM, C), lambda i: (i, 0)),
        out_shape=jax.ShapeDtypeStruct((N, C), jnp.float32),
    )(aggp, g1, degp, b1)


def kernel(x, edge_index, W0, b0, W1, b1):
    src = edge_index[0].astype(jnp.int32)
    dst = edge_index[1].astype(jnp.int32)
    npad = E_PAD - E
    src2d = jnp.concatenate(
        [src, jnp.zeros((npad,), jnp.int32)]).reshape(E_PAD // CHUNK, CHUNK)
    dst2d = jnp.concatenate(
        [dst, jnp.full((npad,), N_PAD - 1, jnp.int32)]).reshape(
            E_PAD // CHUNK, CHUNK)
    zeros_rows = jnp.zeros((N_PAD, H), jnp.float32)
    zeros_n = jnp.zeros((N_PAD,), jnp.float32)

    degp = _sc_degree(dst2d, zeros_n)[:, :N].T
    h0 = _tc_matmul(x, W0)
    g0 = _tc_scale(h0, degp)
    agg0 = _sc_aggregate(g0, src2d, dst2d, zeros_rows)
    g1 = _tc_mid(agg0, g0, degp, b0.reshape(1, H), W1)
    agg1 = _sc_aggregate(g1, src2d, dst2d, zeros_rows)
    return _tc_final(agg1, g1, degp, b1.reshape(1, C))


# speedup vs baseline: 23.9174x; 1.8362x over previous
import functools

import jax
import jax.numpy as jnp
from jax import lax
from jax.experimental import pallas as pl
from jax.experimental.pallas import tpu as pltpu
from jax.experimental.pallas import tpu_sc as plsc

N = 10000
E = 160000
D = 256
H = 64
C = 64

NC = 2
NS = 16
CHUNK = 128
E_PAD = 163840
N_PAD = 10240
CHUNKS_PER_TILE = (E_PAD // CHUNK) // (NC * NS)
ROWS_PER_TILE = N_PAD // NS

_MESH = dict(core_axis_name="c", subcore_axis_name="s")


def _sc_degree(dst2d, zeros_n):
    mesh = plsc.VectorSubcoreMesh(**_MESH)

    @functools.partial(
        pl.kernel,
        out_type=jax.ShapeDtypeStruct((NC, N_PAD), jnp.float32),
        mesh=mesh,
        compiler_params=pltpu.CompilerParams(use_tc_tiling_on_sc=False),
        scratch_types=[
            pltpu.VMEM((CHUNKS_PER_TILE, CHUNK), jnp.int32),
            pltpu.VMEM((CHUNK,), jnp.float32),
            pltpu.VMEM_SHARED((N_PAD,), jnp.float32),
        ],
    )
    def deg_kernel(dst_hbm, zeros_hbm, out_hbm, idx_v, ones_v, acc_sh):
        cid = lax.axis_index("c")
        sid = lax.axis_index("s")
        wid = cid * NS + sid
        pltpu.sync_copy(dst_hbm.at[:, wid], idx_v)

        @pl.loop(0, CHUNK, step=16)
        def _(i):
            ones_v[pl.ds(i, 16)] = jnp.ones((16,), jnp.float32)

        r0 = sid * ROWS_PER_TILE
        pltpu.sync_copy(zeros_hbm.at[pl.ds(r0, ROWS_PER_TILE)],
                        acc_sh.at[pl.ds(r0, ROWS_PER_TILE)])
        plsc.subcore_barrier()

        @pl.loop(0, CHUNKS_PER_TILE)
        def _(j):
            pltpu.sync_copy(ones_v, acc_sh.at[idx_v.at[j]], add=True)

        plsc.subcore_barrier()
        pltpu.sync_copy(acc_sh.at[pl.ds(r0, ROWS_PER_TILE)],
                        out_hbm.at[cid, pl.ds(r0, ROWS_PER_TILE)])

    return deg_kernel(dst2d, zeros_n)


def _sc_aggregate(g, src2d, dst2d, zeros_rows):
    mesh = plsc.VectorSubcoreMesh(**_MESH)

    @functools.partial(
        pl.kernel,
        out_type=jax.ShapeDtypeStruct((NC, N_PAD, H), jnp.float32),
        mesh=mesh,
        compiler_params=pltpu.CompilerParams(use_tc_tiling_on_sc=False),
        scratch_types=[
            pltpu.VMEM((CHUNKS_PER_TILE, CHUNK), jnp.int32),
            pltpu.VMEM((CHUNKS_PER_TILE, CHUNK), jnp.int32),
            pltpu.VMEM((CHUNK, H), jnp.float32),
            pltpu.VMEM((CHUNK, H), jnp.float32),
            pltpu.VMEM_SHARED((N_PAD, H), jnp.float32),
            pltpu.SemaphoreType.DMA,
            pltpu.SemaphoreType.DMA,
        ],
    )
    def agg_kernel(g_hbm, src_hbm, dst_hbm, zeros_hbm, out_hbm,
                   src_v, dst_v, rows_a, rows_b, acc_sh, sem_a, sem_b):
        cid = lax.axis_index("c")
        sid = lax.axis_index("s")
        wid = cid * NS + sid
        pltpu.sync_copy(src_hbm.at[:, wid], src_v)
        pltpu.sync_copy(dst_hbm.at[:, wid], dst_v)

        r0 = sid * ROWS_PER_TILE
        pltpu.sync_copy(zeros_hbm.at[pl.ds(r0, ROWS_PER_TILE)],
                        acc_sh.at[pl.ds(r0, ROWS_PER_TILE)])
        plsc.subcore_barrier()

        pltpu.async_copy(g_hbm.at[src_v.at[0]], rows_a, sem_a)

        @pl.loop(0, CHUNKS_PER_TILE, step=2)
        def _(j):
            pltpu.async_copy(g_hbm.at[src_v.at[j + 1]], rows_b, sem_b)
            pltpu.make_async_copy(g_hbm.at[src_v.at[j]], rows_a, sem_a).wait()
            pltpu.sync_copy(rows_a, acc_sh.at[dst_v.at[j]], add=True)

            @pl.when(j + 2 < CHUNKS_PER_TILE)
            def _():
                pltpu.async_copy(g_hbm.at[src_v.at[j + 2]], rows_a, sem_a)

            pltpu.make_async_copy(
                g_hbm.at[src_v.at[j + 1]], rows_b, sem_b).wait()
            pltpu.sync_copy(rows_b, acc_sh.at[dst_v.at[j + 1]], add=True)

        plsc.subcore_barrier()
        pltpu.sync_copy(acc_sh.at[pl.ds(r0, ROWS_PER_TILE)],
                        out_hbm.at[cid, pl.ds(r0, ROWS_PER_TILE)])

    return agg_kernel(g, src2d, dst2d, zeros_rows)


_BM = 1000


def _dot(a, b):
    return lax.dot_general(a, b, (((1,), (0,)), ((), ())),
                           precision=lax.Precision.HIGHEST,
                           preferred_element_type=jnp.float32)


def _tc_matmul(x, w):
    m, k = x.shape
    _, h = w.shape

    def body(x_ref, w_ref, o_ref):
        o_ref[...] = _dot(x_ref[...], w_ref[...])

    return pl.pallas_call(
        body,
        grid=(m // _BM,),
        in_specs=[pl.BlockSpec((_BM, k), lambda i: (i, 0)),
                  pl.BlockSpec((k, h), lambda i: (0, 0))],
        out_specs=pl.BlockSpec((_BM, h), lambda i: (i, 0)),
        out_shape=jax.ShapeDtypeStruct((m, h), jnp.float32),
    )(x, w)


def _dinv_of(d_ref):
    return lax.rsqrt(d_ref[:, 0:1] + d_ref[:, 1:2] + 1.0)


_DEG_SPEC = pl.BlockSpec((_BM, 2), lambda i: (i, 0))


def _tc_scale(h, degp):
    def body(h_ref, d_ref, o_ref):
        o_ref[...] = h_ref[...] * _dinv_of(d_ref)

    return pl.pallas_call(
        body,
        grid=(N // _BM,),
        in_specs=[pl.BlockSpec((_BM, H), lambda i: (i, 0)),
                  _DEG_SPEC],
        out_specs=pl.BlockSpec((_BM, H), lambda i: (i, 0)),
        out_shape=jax.ShapeDtypeStruct((N, H), jnp.float32),
    )(h, degp)


def _tc_mid(aggp, g0, degp, b0, w1):
    def body(a_ref, g_ref, d_ref, b_ref, w_ref, o_ref):
        dinv = _dinv_of(d_ref)
        s = (a_ref[0] + a_ref[1] + g_ref[...]) * dinv + b_ref[...]
        o_ref[...] = _dot(jnp.maximum(s, 0.0), w_ref[...]) * dinv

    return pl.pallas_call(
        body,
        grid=(N // _BM,),
        in_specs=[pl.BlockSpec((2, _BM, H), lambda i: (0, i, 0),),
                  pl.BlockSpec((_BM, H), lambda i: (i, 0)),
                  _DEG_SPEC,
                  pl.BlockSpec((1, H), lambda i: (0, 0)),
                  pl.BlockSpec((H, C), lambda i: (0, 0))],
        out_specs=pl.BlockSpec((_BM, C), lambda i: (i, 0)),
        out_shape=jax.ShapeDtypeStruct((N, C), jnp.float32),
    )(aggp, g0, degp, b0, w1)


def _tc_final(aggp, g1, degp, b1):
    def body(a_ref, g_ref, d_ref, b_ref, o_ref):
        dinv = _dinv_of(d_ref)
        o_ref[...] = (a_ref[0] + a_ref[1] + g_ref[...]) * dinv + b_ref[...]

    return pl.pallas_call(
        body,
        grid=(N // _BM,),
        in_specs=[pl.BlockSpec((2, _BM, C), lambda i: (0, i, 0)),
                  pl.BlockSpec((_BM, C), lambda i: (i, 0)),
                  _DEG_SPEC,
                  pl.BlockSpec((1, C), lambda i: (0, 0))],
        out_specs=pl.BlockSpec((_BM, C), lambda i: (i, 0)),
        out_shape=jax.ShapeDtypeStruct((N, C), jnp.float32),
    )(aggp, g1, degp, b1)


def kernel(x, edge_index, W0, b0, W1, b1):
    src = edge_index[0].astype(jnp.int32)
    dst = edge_index[1].astype(jnp.int32)
    npad = E_PAD - E
    pad_ix = jnp.arange(npad, dtype=jnp.int32)
    src2d = jnp.concatenate([src, pad_ix % N]).reshape(
        CHUNKS_PER_TILE, NC * NS, CHUNK)
    dst2d = jnp.concatenate([dst, N + pad_ix % (N_PAD - N)]).reshape(
        CHUNKS_PER_TILE, NC * NS, CHUNK)
    zeros_rows = jnp.zeros((N_PAD, H), jnp.float32)
    zeros_n = jnp.zeros((N_PAD,), jnp.float32)

    degp = _sc_degree(dst2d, zeros_n)[:, :N].T
    h0 = _tc_matmul(x, W0)
    g0 = _tc_scale(h0, degp)
    agg0 = _sc_aggregate(g0, src2d, dst2d, zeros_rows)
    g1 = _tc_mid(agg0, g0, degp, b0.reshape(1, H), W1)
    agg1 = _sc_aggregate(g1, src2d, dst2d, zeros_rows)
    return _tc_final(agg1, g1, degp, b1.reshape(1, C))


# speedup vs baseline: 24.0808x; 1.0068x over previous
import functools

import jax
import jax.numpy as jnp
from jax import lax
from jax.experimental import pallas as pl
from jax.experimental.pallas import tpu as pltpu
from jax.experimental.pallas import tpu_sc as plsc

N = 10000
E = 160000
D = 256
H = 64
C = 64

NC = 2
NS = 16
CHUNK = 128
E_PAD = 163840
N_PAD = 10240
CHUNKS_PER_TILE = (E_PAD // CHUNK) // (NC * NS)
ROWS_PER_TILE = N_PAD // NS

_MESH = dict(core_axis_name="c", subcore_axis_name="s")


def _sc_degree(dst2d, zeros_n):
    mesh = plsc.VectorSubcoreMesh(**_MESH)

    @functools.partial(
        pl.kernel,
        out_type=jax.ShapeDtypeStruct((NC, N_PAD), jnp.float32),
        mesh=mesh,
        compiler_params=pltpu.CompilerParams(use_tc_tiling_on_sc=False),
        scratch_types=[
            pltpu.VMEM((CHUNKS_PER_TILE, CHUNK), jnp.int32),
            pltpu.VMEM((CHUNK,), jnp.float32),
            pltpu.VMEM_SHARED((N_PAD,), jnp.float32),
        ],
    )
    def deg_kernel(dst_hbm, zeros_hbm, out_hbm, idx_v, ones_v, acc_sh):
        cid = lax.axis_index("c")
        sid = lax.axis_index("s")
        wid = cid * NS + sid
        pltpu.sync_copy(dst_hbm.at[:, wid], idx_v)

        @pl.loop(0, CHUNK, step=16)
        def _(i):
            ones_v[pl.ds(i, 16)] = jnp.ones((16,), jnp.float32)

        r0 = sid * ROWS_PER_TILE
        pltpu.sync_copy(zeros_hbm, acc_sh.at[pl.ds(r0, ROWS_PER_TILE)])
        plsc.subcore_barrier()

        @pl.loop(0, CHUNKS_PER_TILE)
        def _(j):
            pltpu.sync_copy(ones_v, acc_sh.at[idx_v.at[j]], add=True)

        plsc.subcore_barrier()
        pltpu.sync_copy(acc_sh.at[pl.ds(r0, ROWS_PER_TILE)],
                        out_hbm.at[cid, pl.ds(r0, ROWS_PER_TILE)])

    return deg_kernel(dst2d, zeros_n)


def _sc_aggregate(g, src2d, dst2d, zeros_rows):
    mesh = plsc.VectorSubcoreMesh(**_MESH)

    @functools.partial(
        pl.kernel,
        out_type=jax.ShapeDtypeStruct((NC, N_PAD, H), jnp.float32),
        mesh=mesh,
        compiler_params=pltpu.CompilerParams(use_tc_tiling_on_sc=False),
        scratch_types=[
            pltpu.VMEM((CHUNKS_PER_TILE, CHUNK), jnp.int32),
            pltpu.VMEM((CHUNKS_PER_TILE, CHUNK), jnp.int32),
            pltpu.VMEM((CHUNK, H), jnp.float32),
            pltpu.VMEM((CHUNK, H), jnp.float32),
            pltpu.VMEM_SHARED((N_PAD, H), jnp.float32),
            pltpu.SemaphoreType.DMA,
            pltpu.SemaphoreType.DMA,
        ],
    )
    def agg_kernel(g_hbm, src_hbm, dst_hbm, zeros_hbm, out_hbm,
                   src_v, dst_v, rows_a, rows_b, acc_sh, sem_a, sem_b):
        cid = lax.axis_index("c")
        sid = lax.axis_index("s")
        wid = cid * NS + sid
        pltpu.sync_copy(src_hbm.at[:, wid], src_v)
        pltpu.sync_copy(dst_hbm.at[:, wid], dst_v)

        r0 = sid * ROWS_PER_TILE
        pltpu.sync_copy(zeros_hbm, rows_a)

        @pl.loop(0, ROWS_PER_TILE, step=CHUNK)
        def _(k):
            pltpu.sync_copy(rows_a, acc_sh.at[pl.ds(r0 + k, CHUNK)])

        plsc.subcore_barrier()

        pltpu.async_copy(g_hbm.at[src_v.at[0]], rows_a, sem_a)

        @pl.loop(0, CHUNKS_PER_TILE, step=2)
        def _(j):
            pltpu.async_copy(g_hbm.at[src_v.at[j + 1]], rows_b, sem_b)
            pltpu.make_async_copy(g_hbm.at[src_v.at[j]], rows_a, sem_a).wait()
            pltpu.sync_copy(rows_a, acc_sh.at[dst_v.at[j]], add=True)

            @pl.when(j + 2 < CHUNKS_PER_TILE)
            def _():
                pltpu.async_copy(g_hbm.at[src_v.at[j + 2]], rows_a, sem_a)

            pltpu.make_async_copy(
                g_hbm.at[src_v.at[j + 1]], rows_b, sem_b).wait()
            pltpu.sync_copy(rows_b, acc_sh.at[dst_v.at[j + 1]], add=True)

        plsc.subcore_barrier()
        pltpu.sync_copy(acc_sh.at[pl.ds(r0, ROWS_PER_TILE)],
                        out_hbm.at[cid, pl.ds(r0, ROWS_PER_TILE)])

    return agg_kernel(g, src2d, dst2d, zeros_rows)


_BM = 1000


def _dot(a, b):
    return lax.dot_general(a, b, (((1,), (0,)), ((), ())),
                           precision=lax.Precision.HIGHEST,
                           preferred_element_type=jnp.float32)


def _tc_matmul(x, w):
    m, k = x.shape
    _, h = w.shape

    def body(x_ref, w_ref, o_ref):
        o_ref[...] = _dot(x_ref[...], w_ref[...])

    return pl.pallas_call(
        body,
        grid=(m // _BM,),
        in_specs=[pl.BlockSpec((_BM, k), lambda i: (i, 0)),
                  pl.BlockSpec((k, h), lambda i: (0, 0))],
        out_specs=pl.BlockSpec((_BM, h), lambda i: (i, 0)),
        out_shape=jax.ShapeDtypeStruct((m, h), jnp.float32),
    )(x, w)


def _dinv_of(d_ref):
    return lax.rsqrt(d_ref[:, 0:1] + d_ref[:, 1:2] + 1.0)


_DEG_SPEC = pl.BlockSpec((_BM, 2), lambda i: (i, 0))


def _tc_scale(h, degp):
    def body(h_ref, d_ref, o_ref):
        o_ref[...] = h_ref[...] * _dinv_of(d_ref)

    return pl.pallas_call(
        body,
        grid=(N // _BM,),
        in_specs=[pl.BlockSpec((_BM, H), lambda i: (i, 0)),
                  _DEG_SPEC],
        out_specs=pl.BlockSpec((_BM, H), lambda i: (i, 0)),
        out_shape=jax.ShapeDtypeStruct((N, H), jnp.float32),
    )(h, degp)


def _tc_mid(aggp, g0, degp, b0, w1):
    def body(a_ref, g_ref, d_ref, b_ref, w_ref, o_ref):
        dinv = _dinv_of(d_ref)
        s = (a_ref[0] + a_ref[1] + g_ref[...]) * dinv + b_ref[...]
        o_ref[...] = _dot(jnp.maximum(s, 0.0), w_ref[...]) * dinv

    return pl.pallas_call(
        body,
        grid=(N // _BM,),
        in_specs=[pl.BlockSpec((2, _BM, H), lambda i: (0, i, 0),),
                  pl.BlockSpec((_BM, H), lambda i: (i, 0)),
                  _DEG_SPEC,
                  pl.BlockSpec((1, H), lambda i: (0, 0)),
                  pl.BlockSpec((H, C), lambda i: (0, 0))],
        out_specs=pl.BlockSpec((_BM, C), lambda i: (i, 0)),
        out_shape=jax.ShapeDtypeStruct((N, C), jnp.float32),
    )(aggp, g0, degp, b0, w1)


def _tc_final(aggp, g1, degp, b1):
    def body(a_ref, g_ref, d_ref, b_ref, o_ref):
        dinv = _dinv_of(d_ref)
        o_ref[...] = (a_ref[0] + a_ref[1] + g_ref[...]) * dinv + b_ref[...]

    return pl.pallas_call(
        body,
        grid=(N // _BM,),
        in_specs=[pl.BlockSpec((2, _BM, C), lambda i: (0, i, 0)),
                  pl.BlockSpec((_BM, C), lambda i: (i, 0)),
                  _DEG_SPEC,
                  pl.BlockSpec((1, C), lambda i: (0, 0))],
        out_specs=pl.BlockSpec((_BM, C), lambda i: (i, 0)),
        out_shape=jax.ShapeDtypeStruct((N, C), jnp.float32),
    )(aggp, g1, degp, b1)


def kernel(x, edge_index, W0, b0, W1, b1):
    src = edge_index[0].astype(jnp.int32)
    dst = edge_index[1].astype(jnp.int32)
    npad = E_PAD - E
    pad_ix = jnp.arange(npad, dtype=jnp.int32)
    src2d = jnp.concatenate([src, pad_ix % N]).reshape(
        CHUNKS_PER_TILE, NC * NS, CHUNK)
    dst2d = jnp.concatenate([dst, N + pad_ix % (N_PAD - N)]).reshape(
        CHUNKS_PER_TILE, NC * NS, CHUNK)
    zeros_rows = jnp.zeros((CHUNK, H), jnp.float32)
    zeros_n = jnp.zeros((ROWS_PER_TILE,), jnp.float32)

    degp = _sc_degree(dst2d, zeros_n)[:, :N].T
    h0 = _tc_matmul(x, W0)
    g0 = _tc_scale(h0, degp)
    agg0 = _sc_aggregate(g0, src2d, dst2d, zeros_rows)
    g1 = _tc_mid(agg0, g0, degp, b0.reshape(1, H), W1)
    agg1 = _sc_aggregate(g1, src2d, dst2d, zeros_rows)
    return _tc_final(agg1, g1, degp, b1.reshape(1, C))


# speedup vs baseline: 25.2925x; 1.0503x over previous
import functools

import jax
import jax.numpy as jnp
from jax import lax
from jax.experimental import pallas as pl
from jax.experimental.pallas import tpu as pltpu
from jax.experimental.pallas import tpu_sc as plsc

N = 10000
E = 160000
D = 256
H = 64
C = 64

NC = 2
NS = 16
CHUNK = 128
E_PAD = 163840
N_PAD = 10240
CHUNKS_PER_TILE = (E_PAD // CHUNK) // (NC * NS)
ROWS_PER_TILE = N_PAD // NS

_MESH = dict(core_axis_name="c", subcore_axis_name="s")


def _sc_degree(dst2d, zeros_n):
    mesh = plsc.VectorSubcoreMesh(**_MESH)

    @functools.partial(
        pl.kernel,
        out_type=jax.ShapeDtypeStruct((NC, N_PAD), jnp.float32),
        mesh=mesh,
        compiler_params=pltpu.CompilerParams(use_tc_tiling_on_sc=False),
        scratch_types=[
            pltpu.VMEM((CHUNKS_PER_TILE, CHUNK), jnp.int32),
            pltpu.VMEM((CHUNK,), jnp.float32),
            pltpu.VMEM_SHARED((N_PAD,), jnp.float32),
        ],
    )
    def deg_kernel(dst_hbm, zeros_hbm, out_hbm, idx_v, ones_v, acc_sh):
        cid = lax.axis_index("c")
        sid = lax.axis_index("s")
        wid = cid * NS + sid
        pltpu.sync_copy(dst_hbm.at[:, wid], idx_v)

        @pl.loop(0, CHUNK, step=16)
        def _(i):
            ones_v[pl.ds(i, 16)] = jnp.ones((16,), jnp.float32)

        r0 = sid * ROWS_PER_TILE
        pltpu.sync_copy(zeros_hbm, acc_sh.at[pl.ds(r0, ROWS_PER_TILE)])
        plsc.subcore_barrier()

        @pl.loop(0, CHUNKS_PER_TILE)
        def _(j):
            pltpu.sync_copy(ones_v, acc_sh.at[idx_v.at[j]], add=True)

        plsc.subcore_barrier()
        pltpu.sync_copy(acc_sh.at[pl.ds(r0, ROWS_PER_TILE)],
                        out_hbm.at[cid, pl.ds(r0, ROWS_PER_TILE)])

    return deg_kernel(dst2d, zeros_n)


def _sc_aggregate(g, src2d, dst2d, zeros_rows):
    mesh = plsc.VectorSubcoreMesh(**_MESH)

    @functools.partial(
        pl.kernel,
        out_type=jax.ShapeDtypeStruct((NC, N_PAD, H), jnp.float32),
        mesh=mesh,
        compiler_params=pltpu.CompilerParams(use_tc_tiling_on_sc=False),
        scratch_types=[
            pltpu.VMEM((CHUNKS_PER_TILE, CHUNK), jnp.int32),
            pltpu.VMEM((CHUNKS_PER_TILE, CHUNK), jnp.int32),
            pltpu.VMEM((CHUNK, H), jnp.float32),
            pltpu.VMEM((CHUNK, H), jnp.float32),
            pltpu.VMEM_SHARED((N_PAD, H), jnp.float32),
            pltpu.SemaphoreType.DMA,
            pltpu.SemaphoreType.DMA,
        ],
    )
    def agg_kernel(g_hbm, src_hbm, dst_hbm, zeros_hbm, out_hbm,
                   src_v, dst_v, rows_a, rows_b, acc_sh, sem_a, sem_b):
        cid = lax.axis_index("c")
        sid = lax.axis_index("s")
        wid = cid * NS + sid
        pltpu.sync_copy(src_hbm.at[:, wid], src_v)
        pltpu.sync_copy(dst_hbm.at[:, wid], dst_v)

        r0 = sid * ROWS_PER_TILE
        pltpu.sync_copy(zeros_hbm, rows_a)

        @pl.loop(0, ROWS_PER_TILE, step=CHUNK)
        def _(k):
            pltpu.sync_copy(rows_a, acc_sh.at[pl.ds(r0 + k, CHUNK)])

        plsc.subcore_barrier()

        pltpu.async_copy(g_hbm.at[src_v.at[0]], rows_a, sem_a)

        @pl.loop(0, CHUNKS_PER_TILE, step=2)
        def _(j):
            pltpu.async_copy(g_hbm.at[src_v.at[j + 1]], rows_b, sem_b)
            pltpu.make_async_copy(g_hbm.at[src_v.at[j]], rows_a, sem_a).wait()
            pltpu.sync_copy(rows_a, acc_sh.at[dst_v.at[j]], add=True)

            @pl.when(j + 2 < CHUNKS_PER_TILE)
            def _():
                pltpu.async_copy(g_hbm.at[src_v.at[j + 2]], rows_a, sem_a)

            pltpu.make_async_copy(
                g_hbm.at[src_v.at[j + 1]], rows_b, sem_b).wait()
            pltpu.sync_copy(rows_b, acc_sh.at[dst_v.at[j + 1]], add=True)

        plsc.subcore_barrier()
        pltpu.sync_copy(acc_sh.at[pl.ds(r0, ROWS_PER_TILE)],
                        out_hbm.at[cid, pl.ds(r0, ROWS_PER_TILE)])

    return agg_kernel(g, src2d, dst2d, zeros_rows)


_BM = 2000


def _dot(a, b):
    return lax.dot_general(a, b, (((1,), (0,)), ((), ())),
                           precision=lax.Precision.HIGHEST,
                           preferred_element_type=jnp.float32)


def _tc_matmul(x, w):
    m, k = x.shape
    _, h = w.shape

    def body(x_ref, w_ref, o_ref):
        o_ref[...] = _dot(x_ref[...], w_ref[...])

    return pl.pallas_call(
        body,
        grid=(m // _BM,),
        in_specs=[pl.BlockSpec((_BM, k), lambda i: (i, 0)),
                  pl.BlockSpec((k, h), lambda i: (0, 0))],
        out_specs=pl.BlockSpec((_BM, h), lambda i: (i, 0)),
        out_shape=jax.ShapeDtypeStruct((m, h), jnp.float32),
    )(x, w)


def _dinv_of(d_ref):
    return lax.rsqrt(d_ref[:, 0:1] + d_ref[:, 1:2] + 1.0)


_DEG_SPEC = pl.BlockSpec((_BM, 2), lambda i: (i, 0))


def _tc_scale(h, degp):
    def body(h_ref, d_ref, o_ref):
        o_ref[...] = h_ref[...] * _dinv_of(d_ref)

    return pl.pallas_call(
        body,
        grid=(N // _BM,),
        in_specs=[pl.BlockSpec((_BM, H), lambda i: (i, 0)),
                  _DEG_SPEC],
        out_specs=pl.BlockSpec((_BM, H), lambda i: (i, 0)),
        out_shape=jax.ShapeDtypeStruct((N, H), jnp.float32),
    )(h, degp)


def _tc_mid(aggp, g0, degp, b0, w1):
    def body(a_ref, g_ref, d_ref, b_ref, w_ref, o_ref):
        dinv = _dinv_of(d_ref)
        s = (a_ref[0] + a_ref[1] + g_ref[...]) * dinv + b_ref[...]
        o_ref[...] = _dot(jnp.maximum(s, 0.0), w_ref[...]) * dinv

    return pl.pallas_call(
        body,
        grid=(N // _BM,),
        in_specs=[pl.BlockSpec((2, _BM, H), lambda i: (0, i, 0),),
                  pl.BlockSpec((_BM, H), lambda i: (i, 0)),
                  _DEG_SPEC,
                  pl.BlockSpec((1, H), lambda i: (0, 0)),
                  pl.BlockSpec((H, C), lambda i: (0, 0))],
        out_specs=pl.BlockSpec((_BM, C), lambda i: (i, 0)),
        out_shape=jax.ShapeDtypeStruct((N, C), jnp.float32),
    )(aggp, g0, degp, b0, w1)


def _tc_final(aggp, g1, degp, b1):
    def body(a_ref, g_ref, d_ref, b_ref, o_ref):
        dinv = _dinv_of(d_ref)
        o_ref[...] = (a_ref[0] + a_ref[1] + g_ref[...]) * dinv + b_ref[...]

    return pl.pallas_call(
        body,
        grid=(N // _BM,),
        in_specs=[pl.BlockSpec((2, _BM, C), lambda i: (0, i, 0)),
                  pl.BlockSpec((_BM, C), lambda i: (i, 0)),
                  _DEG_SPEC,
                  pl.BlockSpec((1, C), lambda i: (0, 0))],
        out_specs=pl.BlockSpec((_BM, C), lambda i: (i, 0)),
        out_shape=jax.ShapeDtypeStruct((N, C), jnp.float32),
    )(aggp, g1, degp, b1)


def kernel(x, edge_index, W0, b0, W1, b1):
    src = edge_index[0].astype(jnp.int32)
    dst = edge_index[1].astype(jnp.int32)
    npad = E_PAD - E
    pad_ix = jnp.arange(npad, dtype=jnp.int32)
    src2d = jnp.concatenate([src, pad_ix % N]).reshape(
        CHUNKS_PER_TILE, NC * NS, CHUNK)
    dst2d = jnp.concatenate([dst, N + pad_ix % (N_PAD - N)]).reshape(
        CHUNKS_PER_TILE, NC * NS, CHUNK)
    zeros_rows = jnp.zeros((CHUNK, H), jnp.float32)
    zeros_n = jnp.zeros((ROWS_PER_TILE,), jnp.float32)

    degp = _sc_degree(dst2d, zeros_n)[:, :N].T
    h0 = _tc_matmul(x, W0)
    g0 = _tc_scale(h0, degp)
    agg0 = _sc_aggregate(g0, src2d, dst2d, zeros_rows)
    g1 = _tc_mid(agg0, g0, degp, b0.reshape(1, H), W1)
    agg1 = _sc_aggregate(g1, src2d, dst2d, zeros_rows)
    return _tc_final(agg1, g1, degp, b1.reshape(1, C))


# speedup vs baseline: 29.2673x; 1.1572x over previous
import functools

import jax
import jax.numpy as jnp
from jax import lax
from jax.experimental import pallas as pl
from jax.experimental.pallas import tpu as pltpu
from jax.experimental.pallas import tpu_sc as plsc

N = 10000
E = 160000
D = 256
H = 64
C = 64

NC = 2
NS = 16
CHUNK = 128
E_PAD = 163840
N_PAD = 10240
CHUNKS_PER_TILE = (E_PAD // CHUNK) // (NC * NS)
ROWS_PER_TILE = N_PAD // NS

_MESH = dict(core_axis_name="c", subcore_axis_name="s")


def _sc_degree(dst2d, zeros_n):
    mesh = plsc.VectorSubcoreMesh(**_MESH)

    @functools.partial(
        pl.kernel,
        out_type=jax.ShapeDtypeStruct((NC, N_PAD), jnp.float32),
        mesh=mesh,
        compiler_params=pltpu.CompilerParams(use_tc_tiling_on_sc=False),
        scratch_types=[
            pltpu.VMEM((CHUNKS_PER_TILE, CHUNK), jnp.int32),
            pltpu.VMEM((CHUNK,), jnp.float32),
            pltpu.VMEM_SHARED((N_PAD,), jnp.float32),
        ],
    )
    def deg_kernel(dst_hbm, zeros_hbm, out_hbm, idx_v, ones_v, acc_sh):
        cid = lax.axis_index("c")
        sid = lax.axis_index("s")
        wid = cid * NS + sid
        pltpu.sync_copy(dst_hbm.at[:, wid], idx_v)

        @pl.loop(0, CHUNK, step=16)
        def _(i):
            ones_v[pl.ds(i, 16)] = jnp.ones((16,), jnp.float32)

        r0 = sid * ROWS_PER_TILE
        pltpu.sync_copy(zeros_hbm, acc_sh.at[pl.ds(r0, ROWS_PER_TILE)])
        plsc.subcore_barrier()

        @pl.loop(0, CHUNKS_PER_TILE)
        def _(j):
            pltpu.sync_copy(ones_v, acc_sh.at[idx_v.at[j]], add=True)

        plsc.subcore_barrier()
        pltpu.sync_copy(acc_sh.at[pl.ds(r0, ROWS_PER_TILE)],
                        out_hbm.at[cid, pl.ds(r0, ROWS_PER_TILE)])

    return deg_kernel(dst2d, zeros_n)


def _sc_aggregate(g, src2d, dst2d, zeros_rows):
    mesh = plsc.VectorSubcoreMesh(**_MESH)

    @functools.partial(
        pl.kernel,
        out_type=jax.ShapeDtypeStruct((NC, N_PAD, H), jnp.float32),
        mesh=mesh,
        compiler_params=pltpu.CompilerParams(use_tc_tiling_on_sc=False),
        scratch_types=[
            pltpu.VMEM((CHUNKS_PER_TILE, CHUNK), jnp.int32),
            pltpu.VMEM((CHUNKS_PER_TILE, CHUNK), jnp.int32),
            pltpu.VMEM((CHUNK, H), jnp.float32),
            pltpu.VMEM((CHUNK, H), jnp.float32),
            pltpu.VMEM_SHARED((N_PAD, H), jnp.float32),
            pltpu.SemaphoreType.DMA,
            pltpu.SemaphoreType.DMA,
        ],
    )
    def agg_kernel(g_hbm, src_hbm, dst_hbm, zeros_hbm, out_hbm,
                   src_v, dst_v, rows_a, rows_b, acc_sh, sem_a, sem_b):
        cid = lax.axis_index("c")
        sid = lax.axis_index("s")
        wid = cid * NS + sid
        pltpu.sync_copy(src_hbm.at[:, wid], src_v)
        pltpu.sync_copy(dst_hbm.at[:, wid], dst_v)

        r0 = sid * ROWS_PER_TILE
        pltpu.sync_copy(zeros_hbm, rows_a)

        @pl.loop(0, ROWS_PER_TILE, step=CHUNK)
        def _(k):
            pltpu.sync_copy(rows_a, acc_sh.at[pl.ds(r0 + k, CHUNK)])

        plsc.subcore_barrier()

        pltpu.async_copy(g_hbm.at[src_v.at[0]], rows_a, sem_a)

        @pl.loop(0, CHUNKS_PER_TILE, step=2)
        def _(j):
            pltpu.async_copy(g_hbm.at[src_v.at[j + 1]], rows_b, sem_b)
            pltpu.make_async_copy(g_hbm.at[src_v.at[j]], rows_a, sem_a).wait()
            pltpu.sync_copy(rows_a, acc_sh.at[dst_v.at[j]], add=True)

            @pl.when(j + 2 < CHUNKS_PER_TILE)
            def _():
                pltpu.async_copy(g_hbm.at[src_v.at[j + 2]], rows_a, sem_a)

            pltpu.make_async_copy(
                g_hbm.at[src_v.at[j + 1]], rows_b, sem_b).wait()
            pltpu.sync_copy(rows_b, acc_sh.at[dst_v.at[j + 1]], add=True)

        plsc.subcore_barrier()
        pltpu.sync_copy(acc_sh.at[pl.ds(r0, ROWS_PER_TILE)],
                        out_hbm.at[cid, pl.ds(r0, ROWS_PER_TILE)])

    return agg_kernel(g, src2d, dst2d, zeros_rows)


_BM = 2000


def _dot(a, b):
    return lax.dot_general(a, b, (((1,), (0,)), ((), ())),
                           precision=lax.Precision.HIGHEST,
                           preferred_element_type=jnp.float32)


def _dinv_lo_hi(dlo_ref, dhi_ref):
    lo = lax.rsqrt(dlo_ref[:, 0:1] + dlo_ref[:, 1:2] + 1.0)
    hi = lax.rsqrt(dhi_ref[:, 0:1] + dhi_ref[:, 1:2] + 1.0)
    return lo, hi


def _dinvp_of(dlo_ref, dhi_ref):
    lo, hi = _dinv_lo_hi(dlo_ref, dhi_ref)
    return jnp.concatenate(
        [jnp.broadcast_to(lo, (_PB, H)), jnp.broadcast_to(hi, (_PB, H))],
        axis=1)


_NB = N // _BM
_PB = _BM // 2
_DEG_LO = pl.BlockSpec((_PB, 2), lambda i: (i, 0))
_DEG_HI = pl.BlockSpec((_PB, 2), lambda i: (i + _NB, 0))
_PK_SPEC = pl.BlockSpec((_PB, 2 * H), lambda i: (i, 0))
_APK_SPEC = pl.BlockSpec((2, _PB, 2 * H), lambda i: (0, i, 0))


def _tc_g0(x, w0, degp):
    def body(xlo_ref, xhi_ref, w_ref, dlo_ref, dhi_ref, o_ref):
        lo, hi = _dinv_lo_hi(dlo_ref, dhi_ref)
        w = w_ref[...]
        o_ref[...] = jnp.concatenate(
            [_dot(xlo_ref[...], w) * lo, _dot(xhi_ref[...], w) * hi], axis=1)

    return pl.pallas_call(
        body,
        grid=(_NB,),
        in_specs=[pl.BlockSpec((_PB, D), lambda i: (i, 0)),
                  pl.BlockSpec((_PB, D), lambda i: (i + _NB, 0)),
                  pl.BlockSpec((D, H), lambda i: (0, 0)),
                  _DEG_LO, _DEG_HI],
        out_specs=_PK_SPEC,
        out_shape=jax.ShapeDtypeStruct((N // 2, 2 * H), jnp.float32),
    )(x, x, w0, degp, degp)


def _tc_mid(aggp, g0p, degp, b0b, w1b):
    def body(a_ref, g_ref, dlo_ref, dhi_ref, b_ref, w_ref, o_ref):
        dinvp = _dinvp_of(dlo_ref, dhi_ref)
        s = (a_ref[0] + a_ref[1] + g_ref[...]) * dinvp + b_ref[...]
        o_ref[...] = _dot(jnp.maximum(s, 0.0), w_ref[...]) * dinvp

    return pl.pallas_call(
        body,
        grid=(_NB,),
        in_specs=[_APK_SPEC, _PK_SPEC, _DEG_LO, _DEG_HI,
                  pl.BlockSpec((1, 2 * H), lambda i: (0, 0)),
                  pl.BlockSpec((2 * H, 2 * H), lambda i: (0, 0))],
        out_specs=_PK_SPEC,
        out_shape=jax.ShapeDtypeStruct((N // 2, 2 * H), jnp.float32),
    )(aggp, g0p, degp, degp, b0b, w1b)


def _tc_final(aggp, g1p, degp, b1):
    def body(a_ref, g_ref, d_ref, b_ref, o_ref):
        sp = a_ref[0] + a_ref[1] + g_ref[...]
        half = jnp.where(pl.program_id(0) < _NB,
                         sp[:, 0:H], sp[:, H:2 * H])
        dinv = lax.rsqrt(d_ref[:, 0:1] + d_ref[:, 1:2] + 1.0)
        o_ref[...] = half * dinv + b_ref[...]

    return pl.pallas_call(
        body,
        grid=(2 * _NB,),
        in_specs=[pl.BlockSpec((2, _PB, 2 * H), lambda j: (0, j % _NB, 0)),
                  pl.BlockSpec((_PB, 2 * H), lambda j: (j % _NB, 0)),
                  pl.BlockSpec((_PB, 2), lambda j: (j, 0)),
                  pl.BlockSpec((1, C), lambda j: (0, 0))],
        out_specs=pl.BlockSpec((_PB, C), lambda j: (j, 0)),
        out_shape=jax.ShapeDtypeStruct((N, C), jnp.float32),
    )(aggp, g1p, degp, b1)


def kernel(x, edge_index, W0, b0, W1, b1):
    src = edge_index[0].astype(jnp.int32)
    dst = edge_index[1].astype(jnp.int32)
    npad = E_PAD - E
    pad_ix = jnp.arange(npad, dtype=jnp.int32)
    half = N // 2

    def chunked(idx):
        return idx.reshape(CHUNKS_PER_TILE, NC * NS, CHUNK)

    def perm(idx):
        return 2 * (idx % half) + idx // half

    dst2d = chunked(jnp.concatenate([dst, N + pad_ix % (N_PAD - N)]))
    srcp2d = chunked(perm(jnp.concatenate([src, pad_ix % N])))
    dstp2d = chunked(jnp.concatenate([perm(dst), N + pad_ix % (N_PAD - N)]))
    zeros_rows = jnp.zeros((CHUNK, H), jnp.float32)
    zeros_n = jnp.zeros((ROWS_PER_TILE,), jnp.float32)
    b0b = jnp.concatenate([b0, b0]).reshape(1, 2 * H)
    w1b = jnp.zeros((2 * H, 2 * H), jnp.float32)
    w1b = w1b.at[:H, :C].set(W1).at[H:, C:].set(W1)

    degp = _sc_degree(dst2d, zeros_n)[:, :N].T
    g0p = _tc_g0(x, W0, degp)
    agg0 = _sc_aggregate(g0p.reshape(N, H), srcp2d, dstp2d, zeros_rows)
    g1p = _tc_mid(agg0.reshape(NC, N_PAD // 2, 2 * H), g0p, degp, b0b, w1b)
    agg1 = _sc_aggregate(g1p.reshape(N, H), srcp2d, dstp2d, zeros_rows)
    return _tc_final(agg1.reshape(NC, N_PAD // 2, 2 * H), g1p, degp,
                     b1.reshape(1, C))


# speedup vs baseline: 29.9581x; 1.0236x over previous
import functools

import jax
import jax.numpy as jnp
from jax import lax
from jax.experimental import pallas as pl
from jax.experimental.pallas import tpu as pltpu
from jax.experimental.pallas import tpu_sc as plsc

N = 10000
E = 160000
D = 256
H = 64
C = 64

NC = 2
NS = 16
CHUNK = 128
E_PAD = 163840
N_PAD = 10240
CHUNKS_PER_TILE = (E_PAD // CHUNK) // (NC * NS)
ROWS_PER_TILE = N_PAD // NS

_MESH = dict(core_axis_name="c", subcore_axis_name="s")


def _sc_degree(dst2d, zeros_n):
    mesh = plsc.VectorSubcoreMesh(**_MESH)

    @functools.partial(
        pl.kernel,
        out_type=jax.ShapeDtypeStruct((NC, N_PAD), jnp.float32),
        mesh=mesh,
        compiler_params=pltpu.CompilerParams(use_tc_tiling_on_sc=False),
        scratch_types=[
            pltpu.VMEM((CHUNKS_PER_TILE, CHUNK), jnp.int32),
            pltpu.VMEM((CHUNK,), jnp.float32),
            pltpu.VMEM_SHARED((N_PAD,), jnp.float32),
        ],
    )
    def deg_kernel(dst_hbm, zeros_hbm, out_hbm, idx_v, ones_v, acc_sh):
        cid = lax.axis_index("c")
        sid = lax.axis_index("s")
        wid = cid * NS + sid
        pltpu.sync_copy(dst_hbm.at[:, wid], idx_v)

        @pl.loop(0, CHUNK, step=16)
        def _(i):
            ones_v[pl.ds(i, 16)] = jnp.ones((16,), jnp.float32)

        r0 = sid * ROWS_PER_TILE
        pltpu.sync_copy(zeros_hbm, acc_sh.at[pl.ds(r0, ROWS_PER_TILE)])
        plsc.subcore_barrier()

        @pl.loop(0, CHUNKS_PER_TILE)
        def _(j):
            pltpu.sync_copy(ones_v, acc_sh.at[idx_v.at[j]], add=True)

        plsc.subcore_barrier()
        pltpu.sync_copy(acc_sh.at[pl.ds(r0, ROWS_PER_TILE)],
                        out_hbm.at[cid, pl.ds(r0, ROWS_PER_TILE)])

    return deg_kernel(dst2d, zeros_n)


def _sc_aggregate(g, src2d, dst2d, zeros_rows):
    mesh = plsc.VectorSubcoreMesh(**_MESH)

    @functools.partial(
        pl.kernel,
        out_type=jax.ShapeDtypeStruct((NC, N_PAD, H), jnp.float32),
        mesh=mesh,
        compiler_params=pltpu.CompilerParams(use_tc_tiling_on_sc=False),
        scratch_types=[
            pltpu.VMEM((CHUNKS_PER_TILE, CHUNK), jnp.int32),
            pltpu.VMEM((CHUNKS_PER_TILE, CHUNK), jnp.int32),
            pltpu.VMEM((CHUNK, H), jnp.float32),
            pltpu.VMEM((CHUNK, H), jnp.float32),
            pltpu.VMEM_SHARED((N_PAD, H), jnp.float32),
            pltpu.SemaphoreType.DMA,
            pltpu.SemaphoreType.DMA,
        ],
    )
    def agg_kernel(g_hbm, src_hbm, dst_hbm, zeros_hbm, out_hbm,
                   src_v, dst_v, rows_a, rows_b, acc_sh, sem_a, sem_b):
        cid = lax.axis_index("c")
        sid = lax.axis_index("s")
        wid = cid * NS + sid
        pltpu.sync_copy(src_hbm.at[:, wid], src_v)
        pltpu.sync_copy(dst_hbm.at[:, wid], dst_v)

        r0 = sid * ROWS_PER_TILE
        pltpu.sync_copy(zeros_hbm, rows_a)

        @pl.loop(0, ROWS_PER_TILE, step=CHUNK)
        def _(k):
            pltpu.sync_copy(rows_a, acc_sh.at[pl.ds(r0 + k, CHUNK)])

        plsc.subcore_barrier()

        pltpu.async_copy(g_hbm.at[src_v.at[0]], rows_a, sem_a)

        @pl.loop(0, CHUNKS_PER_TILE, step=2)
        def _(j):
            pltpu.async_copy(g_hbm.at[src_v.at[j + 1]], rows_b, sem_b)
            pltpu.make_async_copy(g_hbm.at[src_v.at[j]], rows_a, sem_a).wait()
            pltpu.sync_copy(rows_a, acc_sh.at[dst_v.at[j]], add=True)

            @pl.when(j + 2 < CHUNKS_PER_TILE)
            def _():
                pltpu.async_copy(g_hbm.at[src_v.at[j + 2]], rows_a, sem_a)

            pltpu.make_async_copy(
                g_hbm.at[src_v.at[j + 1]], rows_b, sem_b).wait()
            pltpu.sync_copy(rows_b, acc_sh.at[dst_v.at[j + 1]], add=True)

        plsc.subcore_barrier()
        pltpu.sync_copy(acc_sh.at[pl.ds(r0, ROWS_PER_TILE)],
                        out_hbm.at[cid, pl.ds(r0, ROWS_PER_TILE)])

    return agg_kernel(g, src2d, dst2d, zeros_rows)


_BM = 2000


def _dot(a, b):
    ah = a.astype(jnp.bfloat16)
    al = (a - ah.astype(jnp.float32)).astype(jnp.bfloat16)
    bh = b.astype(jnp.bfloat16)
    bl = (b - bh.astype(jnp.float32)).astype(jnp.bfloat16)

    def d(p, q):
        return lax.dot_general(p, q, (((1,), (0,)), ((), ())),
                               preferred_element_type=jnp.float32)

    return d(ah, bh) + d(al, bh) + d(ah, bl)


def _dinv_lo_hi(dlo_ref, dhi_ref):
    lo = lax.rsqrt(dlo_ref[:, 0:1] + dlo_ref[:, 1:2] + 1.0)
    hi = lax.rsqrt(dhi_ref[:, 0:1] + dhi_ref[:, 1:2] + 1.0)
    return lo, hi


def _dinvp_of(dlo_ref, dhi_ref):
    lo, hi = _dinv_lo_hi(dlo_ref, dhi_ref)
    return jnp.concatenate(
        [jnp.broadcast_to(lo, (_PB, H)), jnp.broadcast_to(hi, (_PB, H))],
        axis=1)


_NB = N // _BM
_PB = _BM // 2
_DEG_LO = pl.BlockSpec((_PB, 2), lambda i: (i, 0))
_DEG_HI = pl.BlockSpec((_PB, 2), lambda i: (i + _NB, 0))
_PK_SPEC = pl.BlockSpec((_PB, 2 * H), lambda i: (i, 0))
_APK_SPEC = pl.BlockSpec((2, _PB, 2 * H), lambda i: (0, i, 0))


def _tc_g0(x, w0, degp):
    def body(xlo_ref, xhi_ref, w_ref, dlo_ref, dhi_ref, o_ref):
        lo, hi = _dinv_lo_hi(dlo_ref, dhi_ref)
        w = w_ref[...]
        o_ref[...] = jnp.concatenate(
            [_dot(xlo_ref[...], w) * lo, _dot(xhi_ref[...], w) * hi], axis=1)

    return pl.pallas_call(
        body,
        grid=(_NB,),
        in_specs=[pl.BlockSpec((_PB, D), lambda i: (i, 0)),
                  pl.BlockSpec((_PB, D), lambda i: (i + _NB, 0)),
                  pl.BlockSpec((D, H), lambda i: (0, 0)),
                  _DEG_LO, _DEG_HI],
        out_specs=_PK_SPEC,
        out_shape=jax.ShapeDtypeStruct((N // 2, 2 * H), jnp.float32),
    )(x, x, w0, degp, degp)


def _tc_mid(aggp, g0p, degp, b0b, w1b):
    def body(a_ref, g_ref, dlo_ref, dhi_ref, b_ref, w_ref, o_ref):
        dinvp = _dinvp_of(dlo_ref, dhi_ref)
        s = (a_ref[0] + a_ref[1] + g_ref[...]) * dinvp + b_ref[...]
        o_ref[...] = _dot(jnp.maximum(s, 0.0), w_ref[...]) * dinvp

    return pl.pallas_call(
        body,
        grid=(_NB,),
        in_specs=[_APK_SPEC, _PK_SPEC, _DEG_LO, _DEG_HI,
                  pl.BlockSpec((1, 2 * H), lambda i: (0, 0)),
                  pl.BlockSpec((2 * H, 2 * H), lambda i: (0, 0))],
        out_specs=_PK_SPEC,
        out_shape=jax.ShapeDtypeStruct((N // 2, 2 * H), jnp.float32),
    )(aggp, g0p, degp, degp, b0b, w1b)


def _tc_final(aggp, g1p, degp, b1):
    def body(a_ref, g_ref, d_ref, b_ref, o_ref):
        sp = a_ref[0] + a_ref[1] + g_ref[...]
        half = jnp.where(pl.program_id(0) < _NB,
                         sp[:, 0:H], sp[:, H:2 * H])
        dinv = lax.rsqrt(d_ref[:, 0:1] + d_ref[:, 1:2] + 1.0)
        o_ref[...] = half * dinv + b_ref[...]

    return pl.pallas_call(
        body,
        grid=(2 * _NB,),
        in_specs=[pl.BlockSpec((2, _PB, 2 * H), lambda j: (0, j % _NB, 0)),
                  pl.BlockSpec((_PB, 2 * H), lambda j: (j % _NB, 0)),
                  pl.BlockSpec((_PB, 2), lambda j: (j, 0)),
                  pl.BlockSpec((1, C), lambda j: (0, 0))],
        out_specs=pl.BlockSpec((_PB, C), lambda j: (j, 0)),
        out_shape=jax.ShapeDtypeStruct((N, C), jnp.float32),
    )(aggp, g1p, degp, b1)


def kernel(x, edge_index, W0, b0, W1, b1):
    src = edge_index[0].astype(jnp.int32)
    dst = edge_index[1].astype(jnp.int32)
    npad = E_PAD - E
    pad_ix = jnp.arange(npad, dtype=jnp.int32)
    half = N // 2

    def chunked(idx):
        return idx.reshape(CHUNKS_PER_TILE, NC * NS, CHUNK)

    def perm(idx):
        return 2 * (idx % half) + idx // half

    dst2d = chunked(jnp.concatenate([dst, N + pad_ix % (N_PAD - N)]))
    srcp2d = chunked(perm(jnp.concatenate([src, pad_ix % N])))
    dstp2d = chunked(jnp.concatenate([perm(dst), N + pad_ix % (N_PAD - N)]))
    zeros_rows = jnp.zeros((CHUNK, H), jnp.float32)
    zeros_n = jnp.zeros((ROWS_PER_TILE,), jnp.float32)
    b0b = jnp.concatenate([b0, b0]).reshape(1, 2 * H)
    w1b = jnp.zeros((2 * H, 2 * H), jnp.float32)
    w1b = w1b.at[:H, :C].set(W1).at[H:, C:].set(W1)

    degp = _sc_degree(dst2d, zeros_n)[:, :N].T
    g0p = _tc_g0(x, W0, degp)
    agg0 = _sc_aggregate(g0p.reshape(N, H), srcp2d, dstp2d, zeros_rows)
    g1p = _tc_mid(agg0.reshape(NC, N_PAD // 2, 2 * H), g0p, degp, b0b, w1b)
    agg1 = _sc_aggregate(g1p.reshape(N, H), srcp2d, dstp2d, zeros_rows)
    return _tc_final(agg1.reshape(NC, N_PAD // 2, 2 * H), g1p, degp,
                     b1.reshape(1, C))


# speedup vs baseline: 33.2362x; 1.1094x over previous
import functools

import jax
import jax.numpy as jnp
from jax import lax
from jax.experimental import pallas as pl
from jax.experimental.pallas import tpu as pltpu
from jax.experimental.pallas import tpu_sc as plsc

N = 10000
E = 160000
D = 256
H = 64
C = 64

NC = 2
NS = 16
CHUNK = 128
E_PAD = 163840
N_PAD = 10240
CHUNKS_PER_TILE = (E_PAD // CHUNK) // (NC * NS)
ROWS_PER_TILE = N_PAD // NS

_MESH = dict(core_axis_name="c", subcore_axis_name="s")


def _sc_degree(dst2d, zeros_n):
    mesh = plsc.VectorSubcoreMesh(**_MESH)

    @functools.partial(
        pl.kernel,
        out_type=jax.ShapeDtypeStruct((NC, N_PAD), jnp.float32),
        mesh=mesh,
        compiler_params=pltpu.CompilerParams(use_tc_tiling_on_sc=False),
        scratch_types=[
            pltpu.VMEM((CHUNKS_PER_TILE, CHUNK), jnp.int32),
            pltpu.VMEM((CHUNK,), jnp.float32),
            pltpu.VMEM_SHARED((N_PAD,), jnp.float32),
        ],
    )
    def deg_kernel(dst_hbm, zeros_hbm, out_hbm, idx_v, ones_v, acc_sh):
        cid = lax.axis_index("c")
        sid = lax.axis_index("s")
        wid = cid * NS + sid
        pltpu.sync_copy(dst_hbm.at[:, wid], idx_v)

        @pl.loop(0, CHUNK, step=16)
        def _(i):
            ones_v[pl.ds(i, 16)] = jnp.ones((16,), jnp.float32)

        r0 = sid * ROWS_PER_TILE
        pltpu.sync_copy(zeros_hbm, acc_sh.at[pl.ds(r0, ROWS_PER_TILE)])
        plsc.subcore_barrier()

        @pl.loop(0, CHUNKS_PER_TILE)
        def _(j):
            pltpu.sync_copy(ones_v, acc_sh.at[idx_v.at[j]], add=True)

        plsc.subcore_barrier()
        pltpu.sync_copy(acc_sh.at[pl.ds(r0, ROWS_PER_TILE)],
                        out_hbm.at[cid, pl.ds(r0, ROWS_PER_TILE)])

    return deg_kernel(dst2d, zeros_n)


def _sc_aggregate(g, src2d, dst2d, zeros_rows):
    mesh = plsc.VectorSubcoreMesh(**_MESH)

    @functools.partial(
        pl.kernel,
        out_type=jax.ShapeDtypeStruct((NC, N_PAD, H), jnp.float32),
        mesh=mesh,
        compiler_params=pltpu.CompilerParams(use_tc_tiling_on_sc=False),
        scratch_types=[
            pltpu.VMEM((CHUNKS_PER_TILE, CHUNK), jnp.int32),
            pltpu.VMEM((CHUNKS_PER_TILE, CHUNK), jnp.int32),
            pltpu.VMEM((CHUNK, H), jnp.float32),
            pltpu.VMEM((CHUNK, H), jnp.float32),
            pltpu.VMEM((CHUNK, H), jnp.float32),
            pltpu.VMEM((CHUNK, H), jnp.float32),
            pltpu.VMEM_SHARED((N_PAD, H), jnp.float32),
            pltpu.SemaphoreType.DMA,
            pltpu.SemaphoreType.DMA,
            pltpu.SemaphoreType.DMA,
            pltpu.SemaphoreType.DMA,
        ],
    )
    def agg_kernel(g_hbm, src_hbm, dst_hbm, zeros_hbm, out_hbm,
                   src_v, dst_v, rows_a, rows_b, rows_c, rows_d,
                   acc_sh, sem_a, sem_b, sem_c, sem_d):
        cid = lax.axis_index("c")
        sid = lax.axis_index("s")
        wid = cid * NS + sid
        pltpu.sync_copy(src_hbm.at[:, wid], src_v)
        pltpu.sync_copy(dst_hbm.at[:, wid], dst_v)

        r0 = sid * ROWS_PER_TILE
        pltpu.sync_copy(zeros_hbm, rows_a)

        @pl.loop(0, ROWS_PER_TILE, step=CHUNK)
        def _(k):
            pltpu.sync_copy(rows_a, acc_sh.at[pl.ds(r0 + k, CHUNK)])

        plsc.subcore_barrier()

        bufs = (rows_a, rows_b, rows_c, rows_d)
        sems = (sem_a, sem_b, sem_c, sem_d)
        pltpu.async_copy(g_hbm.at[src_v.at[0]], rows_a, sem_a)
        pltpu.async_copy(g_hbm.at[src_v.at[1]], rows_b, sem_b)
        pltpu.async_copy(g_hbm.at[src_v.at[2]], rows_c, sem_c)

        @pl.loop(0, CHUNKS_PER_TILE, step=4)
        def _(j):
            for k in range(4):
                nxt = j + k + 3
                if k == 0:
                    pltpu.async_copy(g_hbm.at[src_v.at[nxt]], bufs[3], sems[3])
                else:
                    @pl.when(nxt < CHUNKS_PER_TILE)
                    def _():
                        pltpu.async_copy(
                            g_hbm.at[src_v.at[nxt]], bufs[k - 1], sems[k - 1])
                pltpu.make_async_copy(
                    g_hbm.at[src_v.at[j + k]], bufs[k], sems[k]).wait()
                pltpu.sync_copy(bufs[k], acc_sh.at[dst_v.at[j + k]], add=True)

        plsc.subcore_barrier()
        pltpu.sync_copy(acc_sh.at[pl.ds(r0, ROWS_PER_TILE)],
                        out_hbm.at[cid, pl.ds(r0, ROWS_PER_TILE)])

    return agg_kernel(g, src2d, dst2d, zeros_rows)


_BM = 2000


def _dot(a, b):
    ah = a.astype(jnp.bfloat16)
    al = (a - ah.astype(jnp.float32)).astype(jnp.bfloat16)
    bh = b.astype(jnp.bfloat16)
    bl = (b - bh.astype(jnp.float32)).astype(jnp.bfloat16)

    def d(p, q):
        return lax.dot_general(p, q, (((1,), (0,)), ((), ())),
                               preferred_element_type=jnp.float32)

    return d(ah, bh) + d(al, bh) + d(ah, bl)


def _dinv_lo_hi(dlo_ref, dhi_ref):
    lo = lax.rsqrt(dlo_ref[:, 0:1] + dlo_ref[:, 1:2] + 1.0)
    hi = lax.rsqrt(dhi_ref[:, 0:1] + dhi_ref[:, 1:2] + 1.0)
    return lo, hi


def _dinvp_of(dlo_ref, dhi_ref):
    lo, hi = _dinv_lo_hi(dlo_ref, dhi_ref)
    return jnp.concatenate(
        [jnp.broadcast_to(lo, (_PB, H)), jnp.broadcast_to(hi, (_PB, H))],
        axis=1)


_NB = N // _BM
_PB = _BM // 2
_DEG_LO = pl.BlockSpec((_PB, 2), lambda i: (i, 0))
_DEG_HI = pl.BlockSpec((_PB, 2), lambda i: (i + _NB, 0))
_PK_SPEC = pl.BlockSpec((_PB, 2 * H), lambda i: (i, 0))
_APK_SPEC = pl.BlockSpec((2, _PB, 2 * H), lambda i: (0, i, 0))


def _tc_g0(x, w0, degp):
    def body(xlo_ref, xhi_ref, w_ref, dlo_ref, dhi_ref, o_ref):
        lo, hi = _dinv_lo_hi(dlo_ref, dhi_ref)
        w = w_ref[...]
        o_ref[...] = jnp.concatenate(
            [_dot(xlo_ref[...], w) * lo, _dot(xhi_ref[...], w) * hi], axis=1)

    return pl.pallas_call(
        body,
        grid=(_NB,),
        in_specs=[pl.BlockSpec((_PB, D), lambda i: (i, 0)),
                  pl.BlockSpec((_PB, D), lambda i: (i + _NB, 0)),
                  pl.BlockSpec((D, H), lambda i: (0, 0)),
                  _DEG_LO, _DEG_HI],
        out_specs=_PK_SPEC,
        out_shape=jax.ShapeDtypeStruct((N // 2, 2 * H), jnp.float32),
    )(x, x, w0, degp, degp)


def _tc_mid(aggp, g0p, degp, b0b, w1b):
    def body(a_ref, g_ref, dlo_ref, dhi_ref, b_ref, w_ref, o_ref):
        dinvp = _dinvp_of(dlo_ref, dhi_ref)
        s = (a_ref[0] + a_ref[1] + g_ref[...]) * dinvp + b_ref[...]
        o_ref[...] = _dot(jnp.maximum(s, 0.0), w_ref[...]) * dinvp

    return pl.pallas_call(
        body,
        grid=(_NB,),
        in_specs=[_APK_SPEC, _PK_SPEC, _DEG_LO, _DEG_HI,
                  pl.BlockSpec((1, 2 * H), lambda i: (0, 0)),
                  pl.BlockSpec((2 * H, 2 * H), lambda i: (0, 0))],
        out_specs=_PK_SPEC,
        out_shape=jax.ShapeDtypeStruct((N // 2, 2 * H), jnp.float32),
    )(aggp, g0p, degp, degp, b0b, w1b)


def _tc_final(aggp, g1p, degp, b1):
    def body(a_ref, g_ref, d_ref, b_ref, o_ref):
        sp = a_ref[0] + a_ref[1] + g_ref[...]
        half = jnp.where(pl.program_id(0) < _NB,
                         sp[:, 0:H], sp[:, H:2 * H])
        dinv = lax.rsqrt(d_ref[:, 0:1] + d_ref[:, 1:2] + 1.0)
        o_ref[...] = half * dinv + b_ref[...]

    return pl.pallas_call(
        body,
        grid=(2 * _NB,),
        in_specs=[pl.BlockSpec((2, _PB, 2 * H), lambda j: (0, j % _NB, 0)),
                  pl.BlockSpec((_PB, 2 * H), lambda j: (j % _NB, 0)),
                  pl.BlockSpec((_PB, 2), lambda j: (j, 0)),
                  pl.BlockSpec((1, C), lambda j: (0, 0))],
        out_specs=pl.BlockSpec((_PB, C), lambda j: (j, 0)),
        out_shape=jax.ShapeDtypeStruct((N, C), jnp.float32),
    )(aggp, g1p, degp, b1)


def kernel(x, edge_index, W0, b0, W1, b1):
    src = edge_index[0].astype(jnp.int32)
    dst = edge_index[1].astype(jnp.int32)
    npad = E_PAD - E
    pad_ix = jnp.arange(npad, dtype=jnp.int32)
    half = N // 2

    def chunked(idx):
        return idx.reshape(CHUNKS_PER_TILE, NC * NS, CHUNK)

    def perm(idx):
        return 2 * (idx % half) + idx // half

    dst2d = chunked(jnp.concatenate([dst, N + pad_ix % (N_PAD - N)]))
    srcp2d = chunked(perm(jnp.concatenate([src, pad_ix % N])))
    dstp2d = chunked(jnp.concatenate([perm(dst), N + pad_ix % (N_PAD - N)]))
    zeros_rows = jnp.zeros((CHUNK, H), jnp.float32)
    zeros_n = jnp.zeros((ROWS_PER_TILE,), jnp.float32)
    b0b = jnp.concatenate([b0, b0]).reshape(1, 2 * H)
    w1b = jnp.zeros((2 * H, 2 * H), jnp.float32)
    w1b = w1b.at[:H, :C].set(W1).at[H:, C:].set(W1)

    degp = _sc_degree(dst2d, zeros_n)[:, :N].T
    g0p = _tc_g0(x, W0, degp)
    agg0 = _sc_aggregate(g0p.reshape(N, H), srcp2d, dstp2d, zeros_rows)
    g1p = _tc_mid(agg0.reshape(NC, N_PAD // 2, 2 * H), g0p, degp, b0b, w1b)
    agg1 = _sc_aggregate(g1p.reshape(N, H), srcp2d, dstp2d, zeros_rows)
    return _tc_final(agg1.reshape(NC, N_PAD // 2, 2 * H), g1p, degp,
                     b1.reshape(1, C))
